# baseline (speedup 1.0000x reference)
import functools

import jax
import jax.numpy as jnp
from jax import lax
from jax.experimental import pallas as pl

N = 10000
E = 320000
D_IN = 128
NGRAPHS = 16
N_PAD = 10240
EP = E + N

_f32 = jnp.float32
_i32 = jnp.int32


def _leaky(v, slope):
    return jnp.where(v >= 0, v, slope * v)


def _proj_body(x_ref, wl_ref, wr_ref, xl_ref, xr_ref):
    xv = x_ref[...]
    xl_ref[...] = jnp.dot(xv, wl_ref[...], preferred_element_type=_f32)
    xr_ref[...] = jnp.dot(xv, wr_ref[...], preferred_element_type=_f32)


def _alpha_body(xls_ref, xrd_ref, att_ref, a_ref, *, heads):
    m = _leaky(xls_ref[...] + xrd_ref[...], 0.2)
    cols = []
    for h in range(heads):
        cols.append(jnp.sum(m[:, h * 128:(h + 1) * 128]
                            * att_ref[h, :][None, :], axis=1, keepdims=True))
    out = jnp.concatenate(cols, axis=1)
    if heads < 8:
        out = jnp.concatenate(
            [out, jnp.zeros((out.shape[0], 8 - heads), _f32)], axis=1)
    a_ref[...] = out


def _tc3_body(h2_ref, batch_ref, w1_ref, b1f_ref, w2_ref, b2f_ref, o_ref):
    h2 = h2_ref[...]
    oh = (batch_ref[...] == lax.broadcasted_iota(_i32, (1, NGRAPHS), 1)
          ).astype(_f32)
    sums = lax.dot_general(oh, h2, (((0,), (0,)), ((), ())),
                           preferred_element_type=_f32)
    ones = jnp.ones((h2.shape[0], 1), _f32)
    cnts = lax.dot_general(oh, ones, (((0,), (0,)), ((), ())),
                           preferred_element_type=_f32)
    g = sums / jnp.maximum(cnts, 1.0)
    gg = _leaky(jnp.dot(g, w1_ref[...], preferred_element_type=_f32)
                + b1f_ref[...][None, :], 0.01)
    o_ref[...] = (jnp.dot(gg, w2_ref[...], preferred_element_type=_f32)
                  + b2f_ref[...][None, :])


def _project(x, Wl, Wr):
    n, d = x.shape
    f = Wl.shape[1]
    blk = 1024
    return pl.pallas_call(
        _proj_body,
        grid=(n // blk,),
        in_specs=[
            pl.BlockSpec((blk, d), lambda i: (i, 0)),
            pl.BlockSpec((d, f), lambda i: (0, 0)),
            pl.BlockSpec((d, f), lambda i: (0, 0)),
        ],
        out_specs=[
            pl.BlockSpec((blk, f), lambda i: (i, 0)),
            pl.BlockSpec((blk, f), lambda i: (i, 0)),
        ],
        out_shape=[
            jax.ShapeDtypeStruct((n, f), _f32),
            jax.ShapeDtypeStruct((n, f), _f32),
        ],
    )(x, Wl, Wr)


def _edge_alpha(xls, xrd, att, heads):
    ep, f = xls.shape
    blk = 3000
    pad = (-ep) % blk
    if pad:
        xls = jnp.concatenate([xls, jnp.zeros((pad, f), _f32)], axis=0)
        xrd = jnp.concatenate([xrd, jnp.zeros((pad, f), _f32)], axis=0)
    body = functools.partial(_alpha_body, heads=heads)
    a = pl.pallas_call(
        body,
        grid=((ep + pad) // blk,),
        in_specs=[
            pl.BlockSpec((blk, f), lambda i: (i, 0)),
            pl.BlockSpec((blk, f), lambda i: (i, 0)),
            pl.BlockSpec((att.shape[0], 128), lambda i: (0, 0)),
        ],
        out_specs=pl.BlockSpec((blk, 8), lambda i: (i, 0)),
        out_shape=jax.ShapeDtypeStruct((ep + pad, 8), _f32),
    )(xls, xrd, att)
    return a[:ep]


def _gatv2(x_pad, src, dst, Wl, Wr, att, bias, heads, out_ch):
    xl, xr = _project(x_pad, Wl, Wr)
    xl = xl[:N]
    xr = xr[:N]
    xls_all = xl[src]
    xrd_all = xr[dst]
    alpha_all = _edge_alpha(xls_all, xrd_all, att, heads)
    outs = []
    for h in range(heads):
        alpha = alpha_all[:, h]
        amax = jax.ops.segment_max(alpha, dst, num_segments=N,
                                   indices_are_sorted=True)
        amax = jnp.where(jnp.isfinite(amax), amax, 0.0)
        ex = jnp.exp(alpha - amax[dst])
        denom = jax.ops.segment_sum(ex, dst, num_segments=N,
                                    indices_are_sorted=True)
        coef = ex / (denom[dst] + 1e-16)
        xls = xls_all[:, h * out_ch:(h + 1) * out_ch]
        outs.append(jax.ops.segment_sum(xls * coef[:, None], dst,
                                        num_segments=N,
                                        indices_are_sorted=True))
    out = jnp.stack(outs, axis=1)
    return out.reshape(N, heads * out_ch) + bias


def kernel(x, edge_index, batch, W_l1, W_r1, att1, b1, W_l2, W_r2, att2, b2,
           W_fc1, b_fc1, W_fc2, b_fc2):
    loops = jnp.arange(N, dtype=edge_index.dtype)
    src = jnp.concatenate([edge_index[0], loops])
    dst = jnp.concatenate([edge_index[1], loops])
    order = jnp.argsort(dst)
    src = src[order]
    dst = dst[order]

    x_pad = jnp.concatenate([x, jnp.zeros((N_PAD - N, D_IN), _f32)], axis=0)
    h = _gatv2(x_pad, src, dst, W_l1, W_r1, att1, b1, 4, 128)
    h = _leaky(h, 0.01)
    h_pad = jnp.concatenate([h, jnp.zeros((N_PAD - N, 512), _f32)], axis=0)
    h = _gatv2(h_pad, src, dst, W_l2, W_r2, att2, b2, 1, 128)
    h = _leaky(h, 0.01)

    h2_pad = jnp.concatenate([h, jnp.zeros((N_PAD - N, 128), _f32)], axis=0)
    batch_pad = jnp.concatenate(
        [batch, jnp.full((N_PAD - N,), NGRAPHS, jnp.int32)]).reshape(N_PAD, 1)
    out = pl.pallas_call(
        _tc3_body,
        out_shape=jax.ShapeDtypeStruct((NGRAPHS, W_fc2.shape[1]), _f32),
    )(h2_pad, batch_pad, W_fc1, b_fc1, W_fc2, b_fc2)
    return out
